# baseline (speedup 1.0000x reference)
import functools

import jax
import jax.numpy as jnp
from jax import lax
from jax.experimental import pallas as pl
from jax.experimental.pallas import tpu as pltpu
from jax.experimental.pallas import tpu_sc as plsc

_B = 1024
_D_IN = 1024
_N_EMB = 512
_D_OUT = 1024
_D_CAT = _D_IN + _N_EMB

_NNZ_E = 8192
_NNZ_M = 32768

_NS = 16
_L = 16
_ZLEN = 8192

_E_CHUNK = _NNZ_E // _NS
_M_CHUNK = _NNZ_M // _NS
_C_CHUNK = _E_CHUNK + _M_CHUNK
_NCHUNK = _C_CHUNK // 128

_E_HALF_ROWS = _N_EMB // 2
_M_HALF_ROWS = _D_OUT // 2
_E_HALF_EL = _E_HALF_ROWS * _D_IN
_M_HALF_EL = _M_HALF_ROWS * _D_CAT
_M_OFF = _E_HALF_EL
_TRASH = _E_HALF_EL + _M_HALF_EL
_ACC_EL = _TRASH + _L
_ZSLICE = (_E_HALF_EL + _M_HALF_EL) // _NS

_E_ROWS_PER_TILE = _E_HALF_ROWS // _NS
_M_ROWS_PER_TILE = _M_HALF_ROWS // _NS


def _make_densify_both():
  mesh = plsc.VectorSubcoreMesh(core_axis_name="c", subcore_axis_name="s")

  @functools.partial(
      pl.kernel,
      mesh=mesh,
      out_type=(jax.ShapeDtypeStruct((_N_EMB, _D_IN), jnp.float32),
                jax.ShapeDtypeStruct((_D_OUT, _D_CAT), jnp.float32)),
      scratch_types=[
          pltpu.VMEM((_C_CHUNK,), jnp.int32),
          pltpu.VMEM((_C_CHUNK,), jnp.int32),
          pltpu.VMEM((_C_CHUNK,), jnp.float32),
          pltpu.VMEM((_NCHUNK, 128), jnp.int32),
          pltpu.VMEM((_ZLEN,), jnp.float32),
          pltpu.VMEM_SHARED((_ACC_EL,), jnp.float32),
          pltpu.SemaphoreType.DMA,
          pltpu.SemaphoreType.DMA,
          pltpu.SemaphoreType.DMA,
      ],
  )
  def dens(erows_hbm, ecols_hbm, evals_hbm, mrows_hbm, mcols_hbm, mvals_hbm,
           eout_hbm, mout_hbm,
           rows_v, cols_v, vals_v, idx_v, z_v, acc_s, sem, zsem, ssem):
    cid = lax.axis_index("c")
    sid = lax.axis_index("s")
    ebase = sid * _E_CHUNK
    mbase = sid * _M_CHUNK

    zvec = jnp.zeros((_L,), jnp.float32)

    def zfill(i, carry):
      for u in range(8):
        z_v[pl.ds((i * 8 + u) * _L, _L)] = zvec
      return carry

    lax.fori_loop(0, _ZLEN // (8 * _L), zfill, 0)

    zeros = [
        pltpu.async_copy(
            z_v, acc_s.at[pl.ds(sid * _ZSLICE + i * _ZLEN, _ZLEN)], zsem)
        for i in range(_ZSLICE // _ZLEN)
    ]

    loads = [
        pltpu.async_copy(erows_hbm.at[pl.ds(ebase, _E_CHUNK)],
                         rows_v.at[pl.ds(0, _E_CHUNK)], sem),
        pltpu.async_copy(ecols_hbm.at[pl.ds(ebase, _E_CHUNK)],
                         cols_v.at[pl.ds(0, _E_CHUNK)], sem),
        pltpu.async_copy(evals_hbm.at[pl.ds(ebase, _E_CHUNK)],
                         vals_v.at[pl.ds(0, _E_CHUNK)], sem),
        pltpu.async_copy(mrows_hbm.at[pl.ds(mbase, _M_CHUNK)],
                         rows_v.at[pl.ds(_E_CHUNK, _M_CHUNK)], sem),
        pltpu.async_copy(mcols_hbm.at[pl.ds(mbase, _M_CHUNK)],
                         cols_v.at[pl.ds(_E_CHUNK, _M_CHUNK)], sem),
        pltpu.async_copy(mvals_hbm.at[pl.ds(mbase, _M_CHUNK)],
                         vals_v.at[pl.ds(_E_CHUNK, _M_CHUNK)], sem),
    ]
    for h in loads:
      h.wait()

    iota = lax.iota(jnp.int32, _L)
    e_row0 = cid * _E_HALF_ROWS
    m_row0 = cid * _M_HALF_ROWS

    def build_e(k, carry):
      for u in range(8):
        j = k * 8 + u
        sl = pl.ds(j * _L, _L)
        r = rows_v[sl]
        c = cols_v[sl]
        mine = (r >= e_row0) & (r < e_row0 + _E_HALF_ROWS)
        loc = (r - e_row0) * _D_IN + c
        idx_v[k, pl.ds(u * _L, _L)] = jnp.where(mine, loc, _TRASH + iota)
      return carry

    lax.fori_loop(0, _E_CHUNK // 128, build_e, 0)

    def build_m(k, carry):
      for u in range(8):
        j = k * 8 + u
        sl = pl.ds(_E_CHUNK + j * _L, _L)
        r = rows_v[sl]
        c = cols_v[sl]
        mine = (r >= m_row0) & (r < m_row0 + _M_HALF_ROWS)
        loc = _M_OFF + (r - m_row0) * _D_CAT + c
        idx_v[_E_CHUNK // 128 + k, pl.ds(u * _L, _L)] = (
            jnp.where(mine, loc, _TRASH + iota))
      return carry

    lax.fori_loop(0, _M_CHUNK // 128, build_m, 0)

    for h in zeros:
      h.wait()
    plsc.subcore_barrier()

    escats = [
        pltpu.async_copy(vals_v.at[pl.ds(k * 128, 128)],
                         acc_s.at[idx_v.at[k]], ssem, add=True)
        for k in range(_E_CHUNK // 128)
    ]
    for h in escats:
      h.wait()
    plsc.subcore_barrier()

    ecopies = []
    for r in range(_E_ROWS_PER_TILE):
      lrow = sid * _E_ROWS_PER_TILE + r
      ecopies.append(pltpu.async_copy(
          acc_s.at[pl.ds(lrow * _D_IN, _D_IN)],
          eout_hbm.at[e_row0 + lrow], sem))

    mscats = [
        pltpu.async_copy(vals_v.at[pl.ds(k * 128, 128)],
                         acc_s.at[idx_v.at[k]], ssem, add=True)
        for k in range(_E_CHUNK // 128, _NCHUNK)
    ]
    for h in mscats:
      h.wait()
    plsc.subcore_barrier()

    mcopies = []
    for r in range(_M_ROWS_PER_TILE):
      lrow = sid * _M_ROWS_PER_TILE + r
      mcopies.append(pltpu.async_copy(
          acc_s.at[pl.ds(_M_OFF + lrow * _D_CAT, _D_CAT)],
          mout_hbm.at[m_row0 + lrow], sem))
    for h in ecopies:
      h.wait()
    for h in mcopies:
      h.wait()

  return dens


_densify_both = _make_densify_both()


_MB = 128


def _tc_body(x_ref, e_ref, m_ref, o_ref, emb_scr):
  @pl.when(pl.program_id(0) == 0)
  def _():
    emb_scr[...] = lax.dot_general(x_ref[...], e_ref[...],
                                   (((1,), (1,)), ((), ())),
                                   preferred_element_type=jnp.float32)

  mb = m_ref[...]
  o_ref[...] = (
      lax.dot_general(x_ref[...], mb[:, :_D_IN], (((1,), (1,)), ((), ())),
                      preferred_element_type=jnp.float32)
      + lax.dot_general(emb_scr[...], mb[:, _D_IN:], (((1,), (1,)), ((), ())),
                        preferred_element_type=jnp.float32))


def _tc_forward(x, e_dense, m_dense):
  return pl.pallas_call(
      _tc_body,
      grid=(_D_OUT // _MB,),
      in_specs=[
          pl.BlockSpec((_B, _D_IN), lambda j: (0, 0)),
          pl.BlockSpec((_N_EMB, _D_IN), lambda j: (0, 0)),
          pl.BlockSpec((_MB, _D_CAT), lambda j: (j, 0)),
      ],
      out_specs=pl.BlockSpec((_B, _MB), lambda j: (0, j)),
      out_shape=jax.ShapeDtypeStruct((_B, _D_OUT), jnp.float32),
      scratch_shapes=[pltpu.VMEM((_B, _N_EMB), jnp.float32)],
  )(x, e_dense, m_dense)


def kernel(input, emb_vals, main_vals, emb_rows, emb_cols, main_rows, main_cols):
  e, m = _densify_both(
      emb_rows.astype(jnp.int32), emb_cols.astype(jnp.int32), emb_vals,
      main_rows.astype(jnp.int32), main_cols.astype(jnp.int32), main_vals)
  return _tc_forward(input, e, m)


# speedup vs baseline: 1.1024x; 1.1024x over previous
import functools

import jax
import jax.numpy as jnp
from jax import lax
from jax.experimental import pallas as pl
from jax.experimental.pallas import tpu as pltpu
from jax.experimental.pallas import tpu_sc as plsc

_B = 1024
_D_IN = 1024
_N_EMB = 512
_D_OUT = 1024
_D_CAT = _D_IN + _N_EMB

_NNZ_E = 8192
_NNZ_M = 32768

_NS = 16
_L = 16
_ZLEN = 8192

_E_CHUNK = _NNZ_E // _NS
_M_CHUNK = _NNZ_M // _NS
_C_CHUNK = _E_CHUNK + _M_CHUNK
_NCHUNK = _C_CHUNK // 128

_E_HALF_ROWS = _N_EMB // 2
_M_HALF_ROWS = _D_OUT // 2
_E_HALF_EL = _E_HALF_ROWS * _D_IN
_M_HALF_EL = _M_HALF_ROWS * _D_CAT
_M_OFF = _E_HALF_EL
_TRASH = _E_HALF_EL + _M_HALF_EL
_ACC_EL = _TRASH + _L
_ZSLICE = (_E_HALF_EL + _M_HALF_EL) // _NS

_E_ROWS_PER_TILE = _E_HALF_ROWS // _NS
_M_ROWS_PER_TILE = _M_HALF_ROWS // _NS


def _make_densify_both():
  mesh = plsc.VectorSubcoreMesh(core_axis_name="c", subcore_axis_name="s")

  @functools.partial(
      pl.kernel,
      mesh=mesh,
      out_type=(jax.ShapeDtypeStruct((_N_EMB, _D_IN), jnp.float32),
                jax.ShapeDtypeStruct((_D_OUT, _D_CAT), jnp.float32)),
      scratch_types=[
          pltpu.VMEM((_C_CHUNK,), jnp.int32),
          pltpu.VMEM((_C_CHUNK,), jnp.int32),
          pltpu.VMEM((_C_CHUNK,), jnp.float32),
          pltpu.VMEM((_NCHUNK, 128), jnp.int32),
          pltpu.VMEM((_ZLEN,), jnp.float32),
          pltpu.VMEM_SHARED((_ACC_EL,), jnp.float32),
          pltpu.SemaphoreType.DMA,
          pltpu.SemaphoreType.DMA,
          pltpu.SemaphoreType.DMA,
      ],
  )
  def dens(erows_hbm, ecols_hbm, evals_hbm, mrows_hbm, mcols_hbm, mvals_hbm,
           eout_hbm, mout_hbm,
           rows_v, cols_v, vals_v, idx_v, z_v, acc_s, sem, zsem, ssem):
    cid = lax.axis_index("c")
    sid = lax.axis_index("s")
    ebase = sid * _E_CHUNK
    mbase = sid * _M_CHUNK

    zvec = jnp.zeros((_L,), jnp.float32)

    def zfill(i, carry):
      for u in range(8):
        z_v[pl.ds((i * 8 + u) * _L, _L)] = zvec
      return carry

    lax.fori_loop(0, _ZLEN // (8 * _L), zfill, 0)

    zeros = [
        pltpu.async_copy(
            z_v, acc_s.at[pl.ds(sid * _ZSLICE + i * _ZLEN, _ZLEN)], zsem)
        for i in range(_ZSLICE // _ZLEN)
    ]

    loads = [
        pltpu.async_copy(erows_hbm.at[pl.ds(ebase, _E_CHUNK)],
                         rows_v.at[pl.ds(0, _E_CHUNK)], sem),
        pltpu.async_copy(ecols_hbm.at[pl.ds(ebase, _E_CHUNK)],
                         cols_v.at[pl.ds(0, _E_CHUNK)], sem),
        pltpu.async_copy(evals_hbm.at[pl.ds(ebase, _E_CHUNK)],
                         vals_v.at[pl.ds(0, _E_CHUNK)], sem),
        pltpu.async_copy(mrows_hbm.at[pl.ds(mbase, _M_CHUNK)],
                         rows_v.at[pl.ds(_E_CHUNK, _M_CHUNK)], sem),
        pltpu.async_copy(mcols_hbm.at[pl.ds(mbase, _M_CHUNK)],
                         cols_v.at[pl.ds(_E_CHUNK, _M_CHUNK)], sem),
        pltpu.async_copy(mvals_hbm.at[pl.ds(mbase, _M_CHUNK)],
                         vals_v.at[pl.ds(_E_CHUNK, _M_CHUNK)], sem),
    ]
    for h in loads:
      h.wait()

    iota = lax.iota(jnp.int32, _L)
    e_row0 = cid * _E_HALF_ROWS
    m_row0 = cid * _M_HALF_ROWS

    def build_e(k, carry):
      for u in range(8):
        j = k * 8 + u
        sl = pl.ds(j * _L, _L)
        r = rows_v[sl]
        c = cols_v[sl]
        mine = (r >= e_row0) & (r < e_row0 + _E_HALF_ROWS)
        loc = (r - e_row0) * _D_IN + c
        idx_v[k, pl.ds(u * _L, _L)] = jnp.where(mine, loc, _TRASH + iota)
      return carry

    lax.fori_loop(0, _E_CHUNK // 128, build_e, 0)

    def build_m(k, carry):
      for u in range(8):
        j = k * 8 + u
        sl = pl.ds(_E_CHUNK + j * _L, _L)
        r = rows_v[sl]
        c = cols_v[sl]
        mine = (r >= m_row0) & (r < m_row0 + _M_HALF_ROWS)
        loc = _M_OFF + (r - m_row0) * _D_CAT + c
        idx_v[_E_CHUNK // 128 + k, pl.ds(u * _L, _L)] = (
            jnp.where(mine, loc, _TRASH + iota))
      return carry

    lax.fori_loop(0, _M_CHUNK // 128, build_m, 0)

    for h in zeros:
      h.wait()
    plsc.subcore_barrier()

    escats = [
        pltpu.async_copy(vals_v.at[pl.ds(k * 128, 128)],
                         acc_s.at[idx_v.at[k]], ssem, add=True)
        for k in range(_E_CHUNK // 128)
    ]
    for h in escats:
      h.wait()
    plsc.subcore_barrier()

    ecopies = []
    for r in range(_E_ROWS_PER_TILE):
      lrow = sid * _E_ROWS_PER_TILE + r
      ecopies.append(pltpu.async_copy(
          acc_s.at[pl.ds(lrow * _D_IN, _D_IN)],
          eout_hbm.at[e_row0 + lrow], sem))

    mscats = [
        pltpu.async_copy(vals_v.at[pl.ds(k * 128, 128)],
                         acc_s.at[idx_v.at[k]], ssem, add=True)
        for k in range(_E_CHUNK // 128, _NCHUNK)
    ]
    for h in mscats:
      h.wait()
    plsc.subcore_barrier()

    mcopies = []
    for r in range(_M_ROWS_PER_TILE):
      lrow = sid * _M_ROWS_PER_TILE + r
      mcopies.append(pltpu.async_copy(
          acc_s.at[pl.ds(_M_OFF + lrow * _D_CAT, _D_CAT)],
          mout_hbm.at[m_row0 + lrow], sem))
    for h in ecopies:
      h.wait()
    for h in mcopies:
      h.wait()

  return dens


_densify_both = _make_densify_both()


def _tc_body(x_ref, e_ref, m_ref, o_ref):
  xb = x_ref[...]
  emb = lax.dot_general(xb, e_ref[...], (((1,), (1,)), ((), ())),
                        preferred_element_type=jnp.float32)
  cat = jnp.concatenate([xb, emb], axis=1)
  o_ref[...] = lax.dot_general(cat, m_ref[...], (((1,), (1,)), ((), ())),
                               preferred_element_type=jnp.float32)


_BB = 256


def _tc_forward(x, e_dense, m_dense):
  return pl.pallas_call(
      _tc_body,
      grid=(_B // _BB,),
      in_specs=[
          pl.BlockSpec((_BB, _D_IN), lambda i: (i, 0)),
          pl.BlockSpec((_N_EMB, _D_IN), lambda i: (0, 0)),
          pl.BlockSpec((_D_OUT, _D_CAT), lambda i: (0, 0)),
      ],
      out_specs=pl.BlockSpec((_BB, _D_OUT), lambda i: (i, 0)),
      out_shape=jax.ShapeDtypeStruct((_B, _D_OUT), jnp.float32),
  )(x, e_dense, m_dense)


def kernel(input, emb_vals, main_vals, emb_rows, emb_cols, main_rows, main_cols):
  e, m = _densify_both(
      emb_rows.astype(jnp.int32), emb_cols.astype(jnp.int32), emb_vals,
      main_rows.astype(jnp.int32), main_cols.astype(jnp.int32), main_vals)
  return _tc_forward(input, e, m)
